# Initial kernel scaffold; baseline (speedup 1.0000x reference)
import jax
import jax.numpy as jnp
from jax.experimental import pallas as pl


def kernel(x, x_e, edge_index, params):
    raise NotImplementedError("write your pallas kernel here")


# speedup vs baseline: 2.3284x; 2.3284x over previous
import functools

import jax
import jax.numpy as jnp
from jax.experimental import pallas as pl
from jax.experimental.pallas import tpu as pltpu

N = 10000
M = 10000
E = 320000
H = 128


def _l2norm(x):
    n = jnp.linalg.norm(x, axis=1, keepdims=True)
    return x / jnp.maximum(n, 1e-12)


def _linear(x, W, b):
    return x @ W.T + b


def _leaky(x):
    return jnp.where(x >= 0, x, 0.01 * x)


def _layer_norm(x, w, b, eps=1e-5):
    m = jnp.mean(x, axis=-1, keepdims=True)
    v = jnp.var(x, axis=-1, keepdims=True)
    return (x - m) / jnp.sqrt(v + eps) * w + b


def _graph_norm(x, w, b, ms, eps):
    mean = jnp.mean(x, axis=0)
    out = x - mean * ms
    var = jnp.mean(out ** 2, axis=0)
    return w * out / jnp.sqrt(var + eps) + b


def _hgconv(x, src, dst, W, b, deg_n_inv, deg_e_inv):
    xl = x @ W.T
    out_e = jax.ops.segment_sum(xl[src], dst, num_segments=M) * deg_e_inv[:, None]
    out_n = jax.ops.segment_sum(out_e[dst], src, num_segments=N) * deg_n_inv[:, None]
    return out_n + b


def _hgconv_dual(x, src, dst, W, b, deg_n_inv, deg_e_inv):
    xl = x @ W.T
    out_e = jax.ops.segment_sum(xl[dst], src, num_segments=N) * deg_n_inv[:, None]
    out_n = jax.ops.segment_sum(out_e[src], dst, num_segments=M) * deg_e_inv[:, None]
    return out_n + b


def _mm_kernel(x_ref, w_ref, b_ref, o_ref):
    o_ref[...] = jnp.dot(x_ref[...], w_ref[...],
                         preferred_element_type=jnp.float32) + b_ref[...]


def _pallas_linear(x, W, b):
    m = x.shape[0]
    return pl.pallas_call(
        _mm_kernel,
        out_shape=jax.ShapeDtypeStruct((m, W.shape[0]), jnp.float32),
    )(x, W.T, b[None, :])


def kernel(x, x_e, edge_index, params):
    p = params
    src = edge_index[0]
    dst = edge_index[1]

    xs = p['x_struct']
    xs = _leaky(_linear(_l2norm(xs), p['in_proj_w'], p['in_proj_b']))
    xn = _leaky(_linear(_l2norm(x), p['n_sem_w'], p['n_sem_b']))
    xe = _leaky(_linear(_l2norm(x_e), p['e_proj_w'], p['e_proj_b']))

    ones_e = jnp.ones((E,), jnp.float32)
    deg_n = jax.ops.segment_sum(ones_e, src, num_segments=N)
    deg_e = jax.ops.segment_sum(ones_e, dst, num_segments=M)
    deg_n_inv = jnp.where(deg_n == 0, 0.0, 1.0 / deg_n)
    deg_e_inv = jnp.where(deg_e == 0, 0.0, 1.0 / deg_e)

    xs = _layer_norm(xs, p['n_norm_w'], p['n_norm_b'])
    xs = _leaky(_hgconv(xs, src, dst, p['hg0_w'], p['hg0_b'], deg_n_inv, deg_e_inv)) \
        + _graph_norm(xs, p['gn_s_w'], p['gn_s_b'], p['gn_s_ms'], float(H))
    xn = _graph_norm(xn, p['gn1_w'], p['gn1_b'], p['gn1_ms'], 1e-5)
    xn = _leaky(_hgconv(xn, src, dst, p['hg1_w'], p['hg1_b'], deg_n_inv, deg_e_inv)) \
        + _linear(xn, p['skip1_w'], p['skip1_b'])
    agg = jax.ops.segment_min(xn[src], dst, num_segments=M)
    xe = _graph_norm(xe + agg, p['gn2_w'], p['gn2_b'], p['gn2_ms'], 1e-5)
    xe = _leaky(_hgconv_dual(xe, src, dst, p['hg2_w'], p['hg2_b'], deg_n_inv, deg_e_inv)) \
        + _linear(xe, p['skip2_w'], p['skip2_b'])
    xf = jnp.concatenate([xs, xn], axis=1)
    xf = _layer_norm(xf, p['nf_ln1_w'], p['nf_ln1_b'])
    xf = _leaky(_linear(xf, p['nf_lin_w'], p['nf_lin_b']))
    xf = _layer_norm(xf, p['nf_ln2_w'], p['nf_ln2_b'])
    xa = jax.ops.segment_min(xf[src], dst, num_segments=M)
    xef = jnp.concatenate([xa, xe], axis=1)
    xef = _layer_norm(xef, p['ef_ln1_w'], p['ef_ln1_b'])
    xef = _leaky(_linear(xef, p['ef_lin_w'], p['ef_lin_b']))
    xef = _layer_norm(xef, p['ef_ln2_w'], p['ef_ln2_b'])
    return _pallas_linear(xef, p['out_w'], p['out_b'])


# speedup vs baseline: 4.0496x; 1.7392x over previous
import functools

import jax
import jax.numpy as jnp
from jax import lax
from jax.experimental import pallas as pl
from jax.experimental.pallas import tpu as pltpu
from jax.experimental.pallas import tpu_sc as plsc

N = 10000
M = 10000
E = 320000
H = 128

_NC = 2
_NS = 16
_NW = _NC * _NS
_K = 80


def _sc_seg_sum_rows(table, gidx, sidx, num_out):
    e_total = gidx.shape[0]
    per_w = e_total // _NW
    n_chunks = per_w // _K
    rows_per_tile = ((num_out + _NS - 1) // _NS + 7) // 8 * 8
    num_pad = rows_per_tile * _NS
    mesh = plsc.VectorSubcoreMesh(core_axis_name="c", subcore_axis_name="s")
    zeros = jnp.zeros((_K, H), jnp.float32)

    chunks = []
    off = 0
    while off < rows_per_tile:
        ln = min(_K, rows_per_tile - off)
        chunks.append((off, ln))
        off += ln

    @functools.partial(
        pl.kernel,
        out_type=jax.ShapeDtypeStruct((_NC * num_pad, H), jnp.float32),
        mesh=mesh,
        scratch_types=[
            pltpu.VMEM((_K,), jnp.int32),
            pltpu.VMEM((_K,), jnp.int32),
            pltpu.VMEM((_K, H), jnp.float32),
            pltpu.VMEM_SHARED((num_pad, H), jnp.float32),
            pltpu.SemaphoreType.DMA,
        ],
    )
    def k(table_h, gidx_h, sidx_h, zero_h, out_h, gi_v, si_v, rows_v, acc_s, sem):
        cid = lax.axis_index("c")
        sid = lax.axis_index("s")
        wid = cid * _NS + sid
        row0 = sid * rows_per_tile
        pltpu.sync_copy(zero_h, rows_v)
        for off, ln in chunks:
            pltpu.sync_copy(rows_v.at[pl.ds(0, ln)], acc_s.at[pl.ds(row0 + off, ln)])
        plsc.subcore_barrier()
        base0 = wid * per_w

        def body(j, carry):
            base = base0 + j * _K
            pltpu.sync_copy(gidx_h.at[pl.ds(base, _K)], gi_v)
            pltpu.sync_copy(sidx_h.at[pl.ds(base, _K)], si_v)
            pltpu.async_copy(table_h.at[gi_v], rows_v, sem).wait()
            pltpu.sync_copy(rows_v, acc_s.at[si_v], add=True)
            return carry

        lax.fori_loop(0, n_chunks, body, 0)
        plsc.subcore_barrier()
        out0 = cid * num_pad + row0
        for off, ln in chunks:
            pltpu.sync_copy(acc_s.at[pl.ds(row0 + off, ln)], rows_v.at[pl.ds(0, ln)])
            pltpu.sync_copy(rows_v.at[pl.ds(0, ln)], out_h.at[pl.ds(out0 + off, ln)])

    out = k(table, gidx, sidx, zeros)
    return out.reshape(_NC, num_pad, H)


def _l2norm(x):
    n = jnp.linalg.norm(x, axis=1, keepdims=True)
    return x / jnp.maximum(n, 1e-12)


def _linear(x, W, b):
    return x @ W.T + b


def _leaky(x):
    return jnp.where(x >= 0, x, 0.01 * x)


def _layer_norm(x, w, b, eps=1e-5):
    m = jnp.mean(x, axis=-1, keepdims=True)
    v = jnp.var(x, axis=-1, keepdims=True)
    return (x - m) / jnp.sqrt(v + eps) * w + b


def _graph_norm(x, w, b, ms, eps):
    mean = jnp.mean(x, axis=0)
    out = x - mean * ms
    var = jnp.mean(out ** 2, axis=0)
    return w * out / jnp.sqrt(var + eps) + b


def _seg_sum(table, gidx, sidx, num_out):
    p = _sc_seg_sum_rows(table, gidx, sidx, num_out)
    return p[0, :num_out] + p[1, :num_out]


def _hgconv(x, src, dst, W, b, deg_n_inv, deg_e_inv):
    xl = x @ W.T
    out_e = _seg_sum(xl, src, dst, M) * deg_e_inv[:, None]
    out_n = _seg_sum(out_e, dst, src, N) * deg_n_inv[:, None]
    return out_n + b


def _hgconv_dual(x, src, dst, W, b, deg_n_inv, deg_e_inv):
    xl = x @ W.T
    out_e = _seg_sum(xl, dst, src, N) * deg_n_inv[:, None]
    out_n = _seg_sum(out_e, src, dst, M) * deg_e_inv[:, None]
    return out_n + b


def _mm_kernel(x_ref, w_ref, b_ref, o_ref):
    o_ref[...] = jnp.dot(x_ref[...], w_ref[...],
                         preferred_element_type=jnp.float32) + b_ref[...]


def _pallas_linear(x, W, b):
    m = x.shape[0]
    return pl.pallas_call(
        _mm_kernel,
        out_shape=jax.ShapeDtypeStruct((m, W.shape[0]), jnp.float32),
    )(x, W.T, b[None, :])


def kernel(x, x_e, edge_index, params):
    p = params
    src = edge_index[0]
    dst = edge_index[1]

    xs = p['x_struct']
    xs = _leaky(_linear(_l2norm(xs), p['in_proj_w'], p['in_proj_b']))
    xn = _leaky(_linear(_l2norm(x), p['n_sem_w'], p['n_sem_b']))
    xe = _leaky(_linear(_l2norm(x_e), p['e_proj_w'], p['e_proj_b']))

    ones_e = jnp.ones((E,), jnp.float32)
    deg_n = jax.ops.segment_sum(ones_e, src, num_segments=N)
    deg_e = jax.ops.segment_sum(ones_e, dst, num_segments=M)
    deg_n_inv = jnp.where(deg_n == 0, 0.0, 1.0 / deg_n)
    deg_e_inv = jnp.where(deg_e == 0, 0.0, 1.0 / deg_e)

    xs = _layer_norm(xs, p['n_norm_w'], p['n_norm_b'])
    xs = _leaky(_hgconv(xs, src, dst, p['hg0_w'], p['hg0_b'], deg_n_inv, deg_e_inv)) \
        + _graph_norm(xs, p['gn_s_w'], p['gn_s_b'], p['gn_s_ms'], float(H))
    xn = _graph_norm(xn, p['gn1_w'], p['gn1_b'], p['gn1_ms'], 1e-5)
    xn = _leaky(_hgconv(xn, src, dst, p['hg1_w'], p['hg1_b'], deg_n_inv, deg_e_inv)) \
        + _linear(xn, p['skip1_w'], p['skip1_b'])
    agg = jax.ops.segment_min(xn[src], dst, num_segments=M)
    xe = _graph_norm(xe + agg, p['gn2_w'], p['gn2_b'], p['gn2_ms'], 1e-5)
    xe = _leaky(_hgconv_dual(xe, src, dst, p['hg2_w'], p['hg2_b'], deg_n_inv, deg_e_inv)) \
        + _linear(xe, p['skip2_w'], p['skip2_b'])
    xf = jnp.concatenate([xs, xn], axis=1)
    xf = _layer_norm(xf, p['nf_ln1_w'], p['nf_ln1_b'])
    xf = _leaky(_linear(xf, p['nf_lin_w'], p['nf_lin_b']))
    xf = _layer_norm(xf, p['nf_ln2_w'], p['nf_ln2_b'])
    xa = jax.ops.segment_min(xf[src], dst, num_segments=M)
    xef = jnp.concatenate([xa, xe], axis=1)
    xef = _layer_norm(xef, p['ef_ln1_w'], p['ef_ln1_b'])
    xef = _leaky(_linear(xef, p['ef_lin_w'], p['ef_lin_b']))
    xef = _layer_norm(xef, p['ef_ln2_w'], p['ef_ln2_b'])
    return _pallas_linear(xef, p['out_w'], p['out_b'])
